# baseline (speedup 1.0000x reference)
import functools

import jax
import jax.numpy as jnp
from jax import lax
from jax.experimental import pallas as pl
from jax.experimental.pallas import tpu as pltpu
from jax.experimental.pallas import tpu_sc as plsc


def kernel(x, emb):
    seq_len = x.shape[1]
    d_model = emb.shape[1]
    info = plsc.get_sparse_core_info()
    nc, ns = info.num_cores, info.num_subcores
    nw = nc * ns
    rows_per_w = seq_len // nw
    mesh = plsc.VectorSubcoreMesh(core_axis_name="c", subcore_axis_name="s")

    ch = 8
    nbuf = 4
    ahead = 1
    nch = rows_per_w // ch

    @functools.partial(
        pl.kernel,
        mesh=mesh,
        out_type=jax.ShapeDtypeStruct((seq_len, d_model), jnp.float32),
        scratch_types=(
            [pltpu.VMEM_SHARED((nbuf, ns, ch, d_model), jnp.float32)]
            + [pltpu.SemaphoreType.DMA] * (2 * nbuf)
        ),
    )
    def copy_k(emb_hbm, out_hbm, buf, *sems):
        sin = sems[:nbuf]
        sout = sems[nbuf:]
        sid = lax.axis_index("s")
        wid = sid * nc + lax.axis_index("c")
        base = wid * rows_per_w

        def start_in(k):
            return pltpu.async_copy(
                emb_hbm.at[pl.ds(base + k * ch, ch)],
                buf.at[k % nbuf, sid],
                sin[k % nbuf],
            )

        def start_out(k):
            return pltpu.async_copy(
                buf.at[k % nbuf, sid],
                out_hbm.at[pl.ds(base + k * ch, ch)],
                sout[k % nbuf],
            )

        in_cp = [None] * nbuf
        out_cp = [None] * nbuf
        for j in range(min(ahead, nch)):
            in_cp[j % nbuf] = start_in(j)
        for i in range(nch):
            s = i % nbuf
            k = i + ahead
            if k < nch:
                sk = k % nbuf
                if out_cp[sk] is not None:
                    out_cp[sk].wait()
                    out_cp[sk] = None
                in_cp[sk] = start_in(k)
            in_cp[s].wait()
            out_cp[s] = start_out(i)
        for s in range(nbuf):
            if out_cp[s] is not None:
                out_cp[s].wait()

    out = copy_k(emb[:seq_len])
    return out[None]


# speedup vs baseline: 1.0070x; 1.0070x over previous
import functools

import jax
import jax.numpy as jnp
from jax import lax
from jax.experimental import pallas as pl
from jax.experimental.pallas import tpu as pltpu
from jax.experimental.pallas import tpu_sc as plsc


def kernel(x, emb):
    seq_len = x.shape[1]
    d_model = emb.shape[1]
    info = plsc.get_sparse_core_info()
    nc, ns = info.num_cores, info.num_subcores
    nw = nc * ns
    rows_per_w = seq_len // nw
    mesh = plsc.VectorSubcoreMesh(core_axis_name="c", subcore_axis_name="s")

    ch = 16
    nbuf = 2
    ahead = 1
    nch = rows_per_w // ch

    @functools.partial(
        pl.kernel,
        mesh=mesh,
        out_type=jax.ShapeDtypeStruct((seq_len, d_model), jnp.float32),
        scratch_types=(
            [pltpu.VMEM_SHARED((nbuf, ns, ch, d_model), jnp.float32)]
            + [pltpu.SemaphoreType.DMA] * (2 * nbuf)
        ),
    )
    def copy_k(emb_hbm, out_hbm, buf, *sems):
        sin = sems[:nbuf]
        sout = sems[nbuf:]
        sid = lax.axis_index("s")
        wid = sid * nc + lax.axis_index("c")
        base = wid * rows_per_w

        def start_in(k):
            return pltpu.async_copy(
                emb_hbm.at[pl.ds(base + k * ch, ch)],
                buf.at[k % nbuf, sid],
                sin[k % nbuf],
            )

        def start_out(k):
            return pltpu.async_copy(
                buf.at[k % nbuf, sid],
                out_hbm.at[pl.ds(base + k * ch, ch)],
                sout[k % nbuf],
            )

        in_cp = [None] * nbuf
        out_cp = [None] * nbuf
        for j in range(min(ahead, nch)):
            in_cp[j % nbuf] = start_in(j)
        for i in range(nch):
            s = i % nbuf
            k = i + ahead
            if k < nch:
                sk = k % nbuf
                if out_cp[sk] is not None:
                    out_cp[sk].wait()
                    out_cp[sk] = None
                in_cp[sk] = start_in(k)
            in_cp[s].wait()
            out_cp[s] = start_out(i)
        for s in range(nbuf):
            if out_cp[s] is not None:
                out_cp[s].wait()

    out = copy_k(emb[:seq_len])
    return out[None]
